# baseline (speedup 1.0000x reference)
import functools

import jax
import jax.numpy as jnp
from jax import lax
from jax.experimental import pallas as pl
from jax.experimental.pallas import tpu as pltpu
from jax.experimental.pallas import tpu_sc as plsc

_NM = 4
_NB = 100000
_D = 64
_B = 16384
_NW = 32
_BPW = _B // _NW
_CH = 128
_NCH = _BPW // _CH
_NST = _NM * _NCH
_BL = 4096
_NBL = -(-_NB // _BL)


def _transform_tc(wv_ref, Wv_ref, out_ref):
    x = wv_ref[0]
    y = Wv_ref[0]
    z = 0.5 + 0.5 * jnp.tanh(0.5 * x)
    s = 0.5 + 0.5 * jnp.tanh(0.5 * y)
    zs = jnp.concatenate((z, z + s - s * z), axis=0)
    out_ref[0] = zs.T


@jax.jit
def _dense_zz(wv, Wv):
    return pl.pallas_call(
        _transform_tc,
        grid=(_NM, _NBL),
        in_specs=[
            pl.BlockSpec((1, _D, _BL), lambda m, c: (m, 0, c)),
            pl.BlockSpec((1, _D, _BL), lambda m, c: (m, 0, c)),
        ],
        out_specs=pl.BlockSpec((1, _BL, 2 * _D), lambda m, c: (m, c, 0)),
        out_shape=jax.ShapeDtypeStruct((_NM, _NB, 2 * _D), jnp.float32),
    )(wv, Wv)


_mesh = plsc.VectorSubcoreMesh(core_axis_name="c", subcore_axis_name="s")


@functools.partial(
    pl.kernel,
    out_type=jax.ShapeDtypeStruct((_NM, _B, 2 * _D), jnp.float32),
    mesh=_mesh,
    compiler_params=pltpu.CompilerParams(use_tc_tiling_on_sc=True),
    scratch_types=[
        pltpu.VMEM((_NCH, _CH), jnp.int32),
        pltpu.VMEM((4, _CH, 2 * _D), jnp.float32),
        pltpu.SemaphoreType.DMA,
        pltpu.SemaphoreType.DMA,
        pltpu.SemaphoreType.DMA,
        pltpu.SemaphoreType.DMA,
        pltpu.SemaphoreType.DMA,
        pltpu.SemaphoreType.DMA,
        pltpu.SemaphoreType.DMA,
        pltpu.SemaphoreType.DMA,
    ],
)
def _lookup_sc(idx_hbm, t_hbm, out_hbm, idx_v, rows,
               g0, g1, g2, g3, o0, o1, o2, o3):
    wid = lax.axis_index("s") * 2 + lax.axis_index("c")
    base = wid * _BPW
    pltpu.sync_copy(idx_hbm.at[wid], idx_v)

    sems_g = (g0, g1, g2, g3)
    sems_o = (o0, o1, o2, o3)

    def src(s):
        m, j = divmod(s, _NCH)
        return t_hbm.at[m].at[idx_v.at[j]]

    def dst(s):
        m, j = divmod(s, _NCH)
        return out_hbm.at[m, pl.ds(base + j * _CH, _CH)]

    for s in range(4):
        pltpu.async_copy(src(s), rows.at[s % 4], sems_g[s % 4])
    for s in range(_NST):
        b = s % 4
        pltpu.make_async_copy(src(s), rows.at[b], sems_g[b]).wait()
        pltpu.async_copy(rows.at[b], dst(s), sems_o[b])
        if s + 4 < _NST:
            pltpu.make_async_copy(rows.at[b], dst(s), sems_o[b]).wait()
            pltpu.async_copy(src(s + 4), rows.at[b], sems_g[b])
    for s in range(_NST - 4, _NST):
        pltpu.make_async_copy(rows.at[s % 4], dst(s), sems_o[s % 4]).wait()


def kernel(box_indices, w, W):
    idx_all = box_indices.astype(jnp.int32).reshape(_NW, _NCH, _CH)
    wv = jnp.swapaxes(w, 1, 2)
    Wv = jnp.swapaxes(W, 1, 2)
    table = _dense_zz(wv, Wv)
    out = _lookup_sc(idx_all, table)
    return out.reshape(_NM, _B, 2, _D)
